# baseline (speedup 1.0000x reference)
import jax
import jax.numpy as jnp
from jax import lax
from jax.experimental import pallas as pl
from jax.experimental.pallas import tpu as pltpu
from jax.experimental.pallas import tpu_sc as plsc

EMB = 128
NG = 32
DEPTH = 2
NSC = 2
NSUB = 16
NW = NSC * NSUB
EK = 80
LL = 16
BE = 1000
BN = 1000

_f32 = jnp.float32


def _bdot(x, w):
    return jnp.dot(x.astype(jnp.bfloat16), w.astype(jnp.bfloat16),
                   preferred_element_type=_f32)


def _hdot(x, w):
    return jnp.dot(x, w, preferred_element_type=_f32,
                   precision=lax.Precision.HIGHEST)


def _edge_he_body(e_ref, w1_ref, b1_ref, w2_ref, b2_ref, o_ref):
    x = e_ref[...]
    for i in range(w1_ref.shape[0]):
        h = jnp.maximum(_bdot(x, w1_ref[i]) + b1_ref[i], 0.0)
        o_ref[i] = _bdot(h, w2_ref[i]) + b2_ref[i]


def _edge_he(eattr, w1s, b1s, w2s, b2s):
    E = eattr.shape[0]
    F = w1s.shape[0]
    mats = pl.BlockSpec((F, EMB, EMB), lambda i: (0, 0, 0))
    vecs = pl.BlockSpec((F, 1, EMB), lambda i: (0, 0, 0))
    return pl.pallas_call(
        _edge_he_body,
        grid=(E // BE,),
        in_specs=[pl.BlockSpec((BE, EMB), lambda i: (i, 0)),
                  mats, vecs, mats, vecs],
        out_specs=pl.BlockSpec((F, BE, EMB), lambda i: (0, i, 0)),
        out_shape=jax.ShapeDtypeStruct((F, E, EMB), _f32),
    )(eattr, w1s, b1s, w2s, b2s)


def _node_hij_body(r_ref, l_ref, wl1, bl1, wl2, bl2, wr1, br1, wr2, br2,
                   hi_ref, hj_ref):
    h = jnp.maximum(_bdot(r_ref[...], wl1[...]) + bl1[...], 0.0)
    hi_ref[...] = _bdot(h, wl2[...]) + bl2[...]
    h2 = jnp.maximum(_bdot(l_ref[...], wr1[...]) + br1[...], 0.0)
    hj_ref[...] = _bdot(h2, wr2[...]) + br2[...]


def _node_hij(right, left, wl1, bl1, wl2, bl2, wr1, br1, wr2, br2):
    N = right.shape[0]
    mat = pl.BlockSpec((EMB, EMB), lambda i: (0, 0))
    vec = pl.BlockSpec((1, EMB), lambda i: (0, 0))
    blk = pl.BlockSpec((BN, EMB), lambda i: (i, 0))
    return pl.pallas_call(
        _node_hij_body,
        grid=(N // BN,),
        in_specs=[blk, blk, mat, vec, mat, vec, mat, vec, mat, vec],
        out_specs=(blk, blk),
        out_shape=(jax.ShapeDtypeStruct((N, EMB), _f32),
                   jax.ShapeDtypeStruct((N, EMB), _f32)),
    )(right, left, wl1, bl1, wl2, bl2, wr1, br1, wr2, br2)


def _msg_body(u_ref, e_ref, we1, be1, we2, be2, w1f, b1f, w2f, b2f, o_ref):
    x = e_ref[...]
    h = jnp.maximum(_bdot(x, we1[...]) + be1[...], 0.0)
    he = _bdot(h, we2[...]) + be2[...]
    t = u_ref[...] + he
    h2 = jnp.maximum(_bdot(t, w1f[...]) + b1f[...], 0.0)
    o_ref[...] = _bdot(h2, w2f[...]) + b2f[...]


def _msg(u, eattr, we1, be1, we2, be2, w1f, b1f, w2f, b2f):
    E = u.shape[0]
    bem = 512
    mat = pl.BlockSpec((EMB, EMB), lambda i: (0, 0))
    vec = pl.BlockSpec((1, EMB), lambda i: (0, 0))
    blk = pl.BlockSpec((bem, EMB), lambda i: (i, 0))
    return pl.pallas_call(
        _msg_body,
        grid=(E // bem,),
        in_specs=[blk, blk, mat, vec, mat, vec, mat, vec, mat, vec],
        out_specs=blk,
        out_shape=jax.ShapeDtypeStruct((E, EMB), _f32),
    )(u, eattr, we1, be1, we2, be2, w1f, b1f, w2f, b2f)


def _post_body(s_ref, r_ref, w1ot, w1ob, b1o, w2o, b2o, o_ref):
    agg = s_ref[0] + s_ref[1]
    pre = _bdot(agg, w1ot[...]) + _bdot(r_ref[...], w1ob[...]) + b1o[...]
    h = jnp.maximum(pre, 0.0)
    o_ref[...] = _bdot(h, w2o[...]) + b2o[...]


def _post(s2, right, w1ot, w1ob, b1o, w2o, b2o):
    N = right.shape[0]
    mat = pl.BlockSpec((EMB, EMB), lambda i: (0, 0))
    vec = pl.BlockSpec((1, EMB), lambda i: (0, 0))
    blk = pl.BlockSpec((BN, EMB), lambda i: (i, 0))
    sblk = pl.BlockSpec((NSC, BN, EMB), lambda i: (0, i, 0))
    return pl.pallas_call(
        _post_body,
        grid=(N // BN,),
        in_specs=[sblk, blk, mat, mat, vec, mat, vec],
        out_specs=blk,
        out_shape=jax.ShapeDtypeStruct((N, EMB), _f32),
    )(s2, right, w1ot, w1ob, b1o, w2o, b2o)


def _gn_stats_body(x_ref, b_ref, s1_ref, s2_ref, c_ref):
    @pl.when(pl.program_id(0) == 0)
    def _():
        s1_ref[...] = jnp.zeros_like(s1_ref)
        s2_ref[...] = jnp.zeros_like(s2_ref)
        c_ref[...] = jnp.zeros_like(c_ref)

    x = x_ref[...]
    g = lax.broadcasted_iota(jnp.int32, (BN, NG), 1).astype(_f32)
    m = (b_ref[...] == g).astype(_f32)
    dn = (((0,), (0,)), ((), ()))
    s1_ref[...] += lax.dot_general(m, x, dn, preferred_element_type=_f32,
                                   precision=lax.Precision.HIGHEST)
    s2_ref[...] += lax.dot_general(m, x * x, dn, preferred_element_type=_f32,
                                   precision=lax.Precision.HIGHEST)
    c_ref[...] += jnp.sum(m, axis=0)[:, None] * jnp.ones((1, EMB), _f32)


def _gn_norm_body(x_ref, b_ref, s1_ref, s2_ref, c_ref, w, bias, msc, o_ref):
    cnt = jnp.maximum(c_ref[...], 1.0)
    mean = s1_ref[...] / cnt
    ms = mean * msc[...]
    var = s2_ref[...] / cnt - 2.0 * ms * mean + ms * ms
    rstd = lax.rsqrt(var + 1e-5)
    g = lax.broadcasted_iota(jnp.int32, (BN, NG), 1).astype(_f32)
    m = (b_ref[...] == g).astype(_f32)
    mloc = _hdot(m, ms)
    rloc = _hdot(m, rstd)
    o_ref[...] = w[...] * (x_ref[...] - mloc) * rloc + bias[...]


def _gn(x, batch_f, w, bias, msc):
    N = x.shape[0]
    blk = pl.BlockSpec((BN, EMB), lambda i: (i, 0))
    bblk = pl.BlockSpec((BN, 1), lambda i: (i, 0))
    stat = pl.BlockSpec((NG, EMB), lambda i: (0, 0))
    vec = pl.BlockSpec((1, EMB), lambda i: (0, 0))
    s1, s2, cnt = pl.pallas_call(
        _gn_stats_body,
        grid=(N // BN,),
        in_specs=[blk, bblk],
        out_specs=(stat, stat, stat),
        out_shape=(jax.ShapeDtypeStruct((NG, EMB), _f32),) * 3,
    )(x, batch_f)
    return pl.pallas_call(
        _gn_norm_body,
        grid=(N // BN,),
        in_specs=[blk, bblk, stat, stat, stat, vec, vec, vec],
        out_specs=blk,
        out_shape=jax.ShapeDtypeStruct((N, EMB), _f32),
    )(x, batch_f, s1, s2, cnt, w, bias, msc)


def _sc_gather(hi, hj, dst3, src3):
    N = hi.shape[0]
    nch = dst3.shape[1]
    ept = nch * EK
    E = NW * ept
    mesh = plsc.VectorSubcoreMesh(core_axis_name="c", subcore_axis_name="s")

    def body(hi_hbm, hj_hbm, dst_hbm, src_hbm, t_hbm,
             dst_v, src_v, bufa, bufb, sga, sgb, st):
        ci = lax.axis_index("c")
        si = lax.axis_index("s")
        wid = si * NSC + ci
        pltpu.sync_copy(dst_hbm.at[wid], dst_v)
        pltpu.sync_copy(src_hbm.at[wid], src_v)

        def fire(j, b):
            return (pltpu.async_copy(hi_hbm.at[dst_v.at[j]], bufa[b], sga[b]),
                    pltpu.async_copy(hj_hbm.at[src_v.at[j]], bufb[b], sgb[b]))

        def compute(b):
            def row(r, rc):
                for l in range(EMB // LL):
                    sl = pl.ds(l * LL, LL)
                    bufa[b][r, sl] = bufa[b][r, sl] + bufb[b][r, sl]
                return rc
            lax.fori_loop(0, EK, row, 0)

        def write_t(j, b):
            base = wid * ept + j * EK
            return pltpu.async_copy(bufa[b], t_hbm.at[pl.ds(base, EK)], st[b])

        def pair(jj, carry):
            j0 = 2 * jj
            j1 = j0 + 1
            g0 = fire(j0, 0)
            g1 = fire(j1, 1)
            for cp in g0:
                cp.wait()
            compute(0)
            t0 = write_t(j0, 0)
            for cp in g1:
                cp.wait()
            compute(1)
            t1 = write_t(j1, 1)
            t0.wait()
            t1.wait()
            return carry
        lax.fori_loop(0, nch // 2, pair, 0)
        if nch % 2:
            g = fire(nch - 1, 0)
            for cp in g:
                cp.wait()
            compute(0)
            write_t(nch - 1, 0).wait()

    kfn = pl.kernel(
        body,
        out_type=jax.ShapeDtypeStruct((E, EMB), _f32),
        mesh=mesh,
        scratch_types=[
            pltpu.VMEM((nch, EK), jnp.int32),
            pltpu.VMEM((nch, EK), jnp.int32),
            [pltpu.VMEM((EK, EMB), _f32)] * 2,
            [pltpu.VMEM((EK, EMB), _f32)] * 2,
            [pltpu.SemaphoreType.DMA] * 2,
            [pltpu.SemaphoreType.DMA] * 2,
            [pltpu.SemaphoreType.DMA] * 2,
        ],
    )
    return kfn(hi, hj, dst3, src3)


def _sc_scatter(msg, dst3, zeros_ne):
    N = zeros_ne.shape[0]
    nch = dst3.shape[1]
    ept = nch * EK
    rps = (N // NSUB) // 8 * 8
    rem = N - NSUB * rps
    rbase = NSUB * rps
    mesh = plsc.VectorSubcoreMesh(core_axis_name="c", subcore_axis_name="s")

    def body(msg_hbm, dst_hbm, z_hbm, out_hbm, s_s, dst_v, bufm, sgm, ssc):
        ci = lax.axis_index("c")
        si = lax.axis_index("s")
        wid = si * NSC + ci
        pltpu.sync_copy(dst_hbm.at[wid], dst_v)
        pltpu.sync_copy(z_hbm.at[pl.ds(si * rps, rps)],
                        s_s.at[pl.ds(si * rps, rps)])
        if rem:
            @pl.when(si == 0)
            def _():
                pltpu.sync_copy(z_hbm.at[pl.ds(rbase, rem)],
                                s_s.at[pl.ds(rbase, rem)])
        plsc.subcore_barrier()

        def read_msg(j, b):
            base = wid * ept + j * EK
            return pltpu.async_copy(msg_hbm.at[pl.ds(base, EK)], bufm[b],
                                    sgm[b])

        def pair(jj, carry):
            j0 = 2 * jj
            j1 = j0 + 1
            m0 = read_msg(j0, 0)
            m1 = read_msg(j1, 1)
            m0.wait()
            s0 = pltpu.async_copy(bufm[0], s_s.at[dst_v.at[j0]], ssc[0],
                                  add=True)
            m1.wait()
            s1 = pltpu.async_copy(bufm[1], s_s.at[dst_v.at[j1]], ssc[1],
                                  add=True)
            s0.wait()
            s1.wait()
            return carry
        lax.fori_loop(0, nch // 2, pair, 0)
        if nch % 2:
            read_msg(nch - 1, 0).wait()
            pltpu.sync_copy(bufm[0], s_s.at[dst_v.at[nch - 1]], add=True)
        plsc.subcore_barrier()
        pltpu.sync_copy(s_s.at[pl.ds(si * rps, rps)],
                        out_hbm.at[ci, pl.ds(si * rps, rps)])
        if rem:
            @pl.when(si == 0)
            def _():
                pltpu.sync_copy(s_s.at[pl.ds(rbase, rem)],
                                out_hbm.at[ci, pl.ds(rbase, rem)])

    kfn = pl.kernel(
        body,
        out_type=jax.ShapeDtypeStruct((NSC, N, EMB), _f32),
        mesh=mesh,
        scratch_types=[
            pltpu.VMEM_SHARED((N, EMB), _f32),
            pltpu.VMEM((nch, EK), jnp.int32),
            [pltpu.VMEM((EK, EMB), _f32)] * 2,
            [pltpu.SemaphoreType.DMA] * 2,
            [pltpu.SemaphoreType.DMA] * 2,
        ],
    )
    return kfn(msg, dst3, zeros_ne)


def _unpack_conv(p):
    w1o = p["out"]["w1"]
    return {
        "wl1": p["left"]["w1"], "bl1": p["left"]["b1"][None, :],
        "wl2": p["left"]["w2"], "bl2": p["left"]["b2"][None, :],
        "wr1": p["right"]["w1"], "br1": p["right"]["b1"][None, :],
        "wr2": p["right"]["w2"], "br2": p["right"]["b2"][None, :],
        "we1": p["edge"]["w1"], "be1": p["edge"]["b1"][None, :],
        "we2": p["edge"]["w2"], "be2": p["edge"]["b2"][None, :],
        "w1f": p["final"]["w1"], "b1f": p["final"]["b1"][None, :],
        "w2f": p["final"]["w2"], "b2f": p["final"]["b2"][None, :],
        "w1ot": w1o[:EMB], "w1ob": w1o[EMB:], "b1o": p["out"]["b1"][None, :],
        "w2o": p["out"]["w2"], "b2o": p["out"]["b2"][None, :],
    }


def kernel(x_constraints, x_variables, edge_index, edge_attr,
           x_constraints_batch, x_variables_batch, params):
    N = x_constraints.shape[0]
    E = edge_attr.shape[0]
    nch = E // NW // EK
    cons_idx = jnp.reshape(edge_index[0], (NW, nch, EK))
    var_idx = jnp.reshape(edge_index[1], (NW, nch, EK))

    convs = [_unpack_conv(params["v_to_c"][0]), _unpack_conv(params["c_to_v"][0]),
             _unpack_conv(params["v_to_c"][1]), _unpack_conv(params["c_to_v"][1])]
    gns = [params["gn_v_to_c"][0], params["gn_c_to_v"][0],
           params["gn_v_to_c"][1], params["gn_c_to_v"][1]]

    zeros_ne = jnp.zeros((N, EMB), _f32)
    cb_f = x_constraints_batch.astype(_f32)[:, None]
    vb_f = x_variables_batch.astype(_f32)[:, None]

    def conv(f, right, left, dst, src):
        hi, hj = _node_hij(right, left, f["wl1"], f["bl1"], f["wl2"], f["bl2"],
                           f["wr1"], f["br1"], f["wr2"], f["br2"])
        u = _sc_gather(hi, hj, dst, src)
        msg = _msg(u, edge_attr, f["we1"], f["be1"], f["we2"], f["be2"],
                   f["w1f"], f["b1f"], f["w2f"], f["b2f"])
        s2 = _sc_scatter(msg, dst, zeros_ne)
        return _post(s2, right, f["w1ot"], f["w1ob"], f["b1o"],
                     f["w2o"], f["b2o"])

    xc, xv = x_constraints, x_variables
    for i in range(DEPTH):
        xc = conv(convs[2 * i], xc, xv, cons_idx, var_idx)
        xv = conv(convs[2 * i + 1], xv, xc, var_idx, cons_idx)
        g = gns[2 * i]
        xc = _gn(xc, cb_f, g["weight"][None, :], g["bias"][None, :],
                 g["mean_scale"][None, :])
        g = gns[2 * i + 1]
        xv = _gn(xv, vb_f, g["weight"][None, :], g["bias"][None, :],
                 g["mean_scale"][None, :])
    return (xc, xv)


# speedup vs baseline: 1.0114x; 1.0114x over previous
import jax
import jax.numpy as jnp
from jax import lax
from jax.experimental import pallas as pl
from jax.experimental.pallas import tpu as pltpu
from jax.experimental.pallas import tpu_sc as plsc

EMB = 128
NG = 32
DEPTH = 2
NSC = 2
NSUB = 16
NW = NSC * NSUB
EK = 80
LL = 16
BE = 1000
BN = 1000

_f32 = jnp.float32


def _bdot(x, w):
    return jnp.dot(x.astype(jnp.bfloat16), w.astype(jnp.bfloat16),
                   preferred_element_type=_f32)


def _hdot(x, w):
    return jnp.dot(x, w, preferred_element_type=_f32,
                   precision=lax.Precision.HIGHEST)


def _edge_he_body(e_ref, w1_ref, b1_ref, w2_ref, b2_ref, o_ref):
    x = e_ref[...]
    for i in range(w1_ref.shape[0]):
        h = jnp.maximum(_bdot(x, w1_ref[i]) + b1_ref[i], 0.0)
        o_ref[i] = _bdot(h, w2_ref[i]) + b2_ref[i]


def _edge_he(eattr, w1s, b1s, w2s, b2s):
    E = eattr.shape[0]
    F = w1s.shape[0]
    mats = pl.BlockSpec((F, EMB, EMB), lambda i: (0, 0, 0))
    vecs = pl.BlockSpec((F, 1, EMB), lambda i: (0, 0, 0))
    return pl.pallas_call(
        _edge_he_body,
        grid=(E // BE,),
        in_specs=[pl.BlockSpec((BE, EMB), lambda i: (i, 0)),
                  mats, vecs, mats, vecs],
        out_specs=pl.BlockSpec((F, BE, EMB), lambda i: (0, i, 0)),
        out_shape=jax.ShapeDtypeStruct((F, E, EMB), _f32),
    )(eattr, w1s, b1s, w2s, b2s)


def _node_hij_body(r_ref, l_ref, wl1, bl1, wl2, bl2, wr1, br1, wr2, br2,
                   hi_ref, hj_ref):
    h = jnp.maximum(_bdot(r_ref[...], wl1[...]) + bl1[...], 0.0)
    hi_ref[...] = _bdot(h, wl2[...]) + bl2[...]
    h2 = jnp.maximum(_bdot(l_ref[...], wr1[...]) + br1[...], 0.0)
    hj_ref[...] = _bdot(h2, wr2[...]) + br2[...]


def _node_hij(right, left, wl1, bl1, wl2, bl2, wr1, br1, wr2, br2):
    N = right.shape[0]
    mat = pl.BlockSpec((EMB, EMB), lambda i: (0, 0))
    vec = pl.BlockSpec((1, EMB), lambda i: (0, 0))
    blk = pl.BlockSpec((BN, EMB), lambda i: (i, 0))
    return pl.pallas_call(
        _node_hij_body,
        grid=(N // BN,),
        in_specs=[blk, blk, mat, vec, mat, vec, mat, vec, mat, vec],
        out_specs=(blk, blk),
        out_shape=(jax.ShapeDtypeStruct((N, EMB), _f32),
                   jax.ShapeDtypeStruct((N, EMB), _f32)),
    )(right, left, wl1, bl1, wl2, bl2, wr1, br1, wr2, br2)


def _msg_body(u_ref, e_ref, we1, be1, we2, be2, w1f, b1f, w2f, b2f, o_ref):
    x = e_ref[...]
    h = jnp.maximum(_bdot(x, we1[...]) + be1[...], 0.0)
    he = _bdot(h, we2[...]) + be2[...]
    t = u_ref[...] + he
    h2 = jnp.maximum(_bdot(t, w1f[...]) + b1f[...], 0.0)
    o_ref[...] = _bdot(h2, w2f[...]) + b2f[...]


def _msg(u, eattr_bf, we1, be1, we2, be2, w1f, b1f, w2f, b2f):
    E = u.shape[0]
    bem = 512
    mat = pl.BlockSpec((EMB, EMB), lambda i: (0, 0))
    vec = pl.BlockSpec((1, EMB), lambda i: (0, 0))
    blk = pl.BlockSpec((bem, EMB), lambda i: (i, 0))
    return pl.pallas_call(
        _msg_body,
        grid=(E // bem,),
        in_specs=[blk, blk, mat, vec, mat, vec, mat, vec, mat, vec],
        out_specs=blk,
        out_shape=jax.ShapeDtypeStruct((E, EMB), _f32),
    )(u, eattr_bf, we1, be1, we2, be2, w1f, b1f, w2f, b2f)


def _post_body(s_ref, r_ref, w1ot, w1ob, b1o, w2o, b2o, o_ref):
    agg = s_ref[0] + s_ref[1]
    pre = _bdot(agg, w1ot[...]) + _bdot(r_ref[...], w1ob[...]) + b1o[...]
    h = jnp.maximum(pre, 0.0)
    o_ref[...] = _bdot(h, w2o[...]) + b2o[...]


def _post(s2, right, w1ot, w1ob, b1o, w2o, b2o):
    N = right.shape[0]
    mat = pl.BlockSpec((EMB, EMB), lambda i: (0, 0))
    vec = pl.BlockSpec((1, EMB), lambda i: (0, 0))
    blk = pl.BlockSpec((BN, EMB), lambda i: (i, 0))
    sblk = pl.BlockSpec((NSC, BN, EMB), lambda i: (0, i, 0))
    return pl.pallas_call(
        _post_body,
        grid=(N // BN,),
        in_specs=[sblk, blk, mat, mat, vec, mat, vec],
        out_specs=blk,
        out_shape=jax.ShapeDtypeStruct((N, EMB), _f32),
    )(s2, right, w1ot, w1ob, b1o, w2o, b2o)


def _gn_stats_body(x_ref, b_ref, s1_ref, s2_ref, c_ref):
    @pl.when(pl.program_id(0) == 0)
    def _():
        s1_ref[...] = jnp.zeros_like(s1_ref)
        s2_ref[...] = jnp.zeros_like(s2_ref)
        c_ref[...] = jnp.zeros_like(c_ref)

    x = x_ref[...]
    g = lax.broadcasted_iota(jnp.int32, (BN, NG), 1).astype(_f32)
    m = (b_ref[...] == g).astype(_f32)
    dn = (((0,), (0,)), ((), ()))
    s1_ref[...] += lax.dot_general(m, x, dn, preferred_element_type=_f32,
                                   precision=lax.Precision.HIGHEST)
    s2_ref[...] += lax.dot_general(m, x * x, dn, preferred_element_type=_f32,
                                   precision=lax.Precision.HIGHEST)
    c_ref[...] += jnp.sum(m, axis=0)[:, None] * jnp.ones((1, EMB), _f32)


def _gn_norm_body(x_ref, b_ref, s1_ref, s2_ref, c_ref, w, bias, msc, o_ref):
    cnt = jnp.maximum(c_ref[...], 1.0)
    mean = s1_ref[...] / cnt
    ms = mean * msc[...]
    var = s2_ref[...] / cnt - 2.0 * ms * mean + ms * ms
    rstd = lax.rsqrt(var + 1e-5)
    g = lax.broadcasted_iota(jnp.int32, (BN, NG), 1).astype(_f32)
    m = (b_ref[...] == g).astype(_f32)
    mloc = _hdot(m, ms)
    rloc = _hdot(m, rstd)
    o_ref[...] = w[...] * (x_ref[...] - mloc) * rloc + bias[...]


def _gn(x, batch_f, w, bias, msc):
    N = x.shape[0]
    blk = pl.BlockSpec((BN, EMB), lambda i: (i, 0))
    bblk = pl.BlockSpec((BN, 1), lambda i: (i, 0))
    stat = pl.BlockSpec((NG, EMB), lambda i: (0, 0))
    vec = pl.BlockSpec((1, EMB), lambda i: (0, 0))
    s1, s2, cnt = pl.pallas_call(
        _gn_stats_body,
        grid=(N // BN,),
        in_specs=[blk, bblk],
        out_specs=(stat, stat, stat),
        out_shape=(jax.ShapeDtypeStruct((NG, EMB), _f32),) * 3,
    )(x, batch_f)
    return pl.pallas_call(
        _gn_norm_body,
        grid=(N // BN,),
        in_specs=[blk, bblk, stat, stat, stat, vec, vec, vec],
        out_specs=blk,
        out_shape=jax.ShapeDtypeStruct((N, EMB), _f32),
    )(x, batch_f, s1, s2, cnt, w, bias, msc)


def _sc_gather(hi, hj, dst3, src3):
    N = hi.shape[0]
    nch = dst3.shape[1]
    ept = nch * EK
    E = NW * ept
    mesh = plsc.VectorSubcoreMesh(core_axis_name="c", subcore_axis_name="s")

    def body(hi_hbm, hj_hbm, dst_hbm, src_hbm, t_hbm,
             dst_v, src_v, bufa, bufb, sga, sgb, st):
        ci = lax.axis_index("c")
        si = lax.axis_index("s")
        wid = si * NSC + ci
        pltpu.sync_copy(dst_hbm.at[wid], dst_v)
        pltpu.sync_copy(src_hbm.at[wid], src_v)

        def fire(j, b):
            return (pltpu.async_copy(hi_hbm.at[dst_v.at[j]], bufa[b], sga[b]),
                    pltpu.async_copy(hj_hbm.at[src_v.at[j]], bufb[b], sgb[b]))

        def compute(b):
            def row(r, rc):
                for l in range(EMB // LL):
                    sl = pl.ds(l * LL, LL)
                    bufa[b][r, sl] = bufa[b][r, sl] + bufb[b][r, sl]
                return rc
            lax.fori_loop(0, EK, row, 0)

        def write_t(j, b):
            base = wid * ept + j * EK
            return pltpu.async_copy(bufa[b], t_hbm.at[pl.ds(base, EK)], st[b])

        def pair(jj, carry):
            j0 = 2 * jj
            j1 = j0 + 1
            g0 = fire(j0, 0)
            g1 = fire(j1, 1)
            for cp in g0:
                cp.wait()
            compute(0)
            t0 = write_t(j0, 0)
            for cp in g1:
                cp.wait()
            compute(1)
            t1 = write_t(j1, 1)
            t0.wait()
            t1.wait()
            return carry
        lax.fori_loop(0, nch // 2, pair, 0)
        if nch % 2:
            g = fire(nch - 1, 0)
            for cp in g:
                cp.wait()
            compute(0)
            write_t(nch - 1, 0).wait()

    kfn = pl.kernel(
        body,
        out_type=jax.ShapeDtypeStruct((E, EMB), _f32),
        mesh=mesh,
        scratch_types=[
            pltpu.VMEM((nch, EK), jnp.int32),
            pltpu.VMEM((nch, EK), jnp.int32),
            [pltpu.VMEM((EK, EMB), _f32)] * 2,
            [pltpu.VMEM((EK, EMB), _f32)] * 2,
            [pltpu.SemaphoreType.DMA] * 2,
            [pltpu.SemaphoreType.DMA] * 2,
            [pltpu.SemaphoreType.DMA] * 2,
        ],
    )
    return kfn(hi, hj, dst3, src3)


def _sc_scatter(msg, dst3, zeros_ne):
    N = zeros_ne.shape[0]
    nch = dst3.shape[1]
    ept = nch * EK
    rps = (N // NSUB) // 8 * 8
    rem = N - NSUB * rps
    rbase = NSUB * rps
    mesh = plsc.VectorSubcoreMesh(core_axis_name="c", subcore_axis_name="s")

    def body(msg_hbm, dst_hbm, z_hbm, out_hbm, s_s, dst_v, bufm, sgm, ssc):
        ci = lax.axis_index("c")
        si = lax.axis_index("s")
        wid = si * NSC + ci
        pltpu.sync_copy(dst_hbm.at[wid], dst_v)
        pltpu.sync_copy(z_hbm.at[pl.ds(si * rps, rps)],
                        s_s.at[pl.ds(si * rps, rps)])
        if rem:
            @pl.when(si == 0)
            def _():
                pltpu.sync_copy(z_hbm.at[pl.ds(rbase, rem)],
                                s_s.at[pl.ds(rbase, rem)])
        plsc.subcore_barrier()

        def read_msg(j, b):
            base = wid * ept + j * EK
            return pltpu.async_copy(msg_hbm.at[pl.ds(base, EK)], bufm[b],
                                    sgm[b])

        def pair(jj, carry):
            j0 = 2 * jj
            j1 = j0 + 1
            m0 = read_msg(j0, 0)
            m1 = read_msg(j1, 1)
            m0.wait()
            s0 = pltpu.async_copy(bufm[0], s_s.at[dst_v.at[j0]], ssc[0],
                                  add=True)
            m1.wait()
            s1 = pltpu.async_copy(bufm[1], s_s.at[dst_v.at[j1]], ssc[1],
                                  add=True)
            s0.wait()
            s1.wait()
            return carry
        lax.fori_loop(0, nch // 2, pair, 0)
        if nch % 2:
            read_msg(nch - 1, 0).wait()
            pltpu.sync_copy(bufm[0], s_s.at[dst_v.at[nch - 1]], add=True)
        plsc.subcore_barrier()
        pltpu.sync_copy(s_s.at[pl.ds(si * rps, rps)],
                        out_hbm.at[ci, pl.ds(si * rps, rps)])
        if rem:
            @pl.when(si == 0)
            def _():
                pltpu.sync_copy(s_s.at[pl.ds(rbase, rem)],
                                out_hbm.at[ci, pl.ds(rbase, rem)])

    kfn = pl.kernel(
        body,
        out_type=jax.ShapeDtypeStruct((NSC, N, EMB), _f32),
        mesh=mesh,
        scratch_types=[
            pltpu.VMEM_SHARED((N, EMB), _f32),
            pltpu.VMEM((nch, EK), jnp.int32),
            [pltpu.VMEM((EK, EMB), _f32)] * 2,
            [pltpu.SemaphoreType.DMA] * 2,
            [pltpu.SemaphoreType.DMA] * 2,
        ],
    )
    return kfn(msg, dst3, zeros_ne)


def _unpack_conv(p):
    w1o = p["out"]["w1"]
    return {
        "wl1": p["left"]["w1"], "bl1": p["left"]["b1"][None, :],
        "wl2": p["left"]["w2"], "bl2": p["left"]["b2"][None, :],
        "wr1": p["right"]["w1"], "br1": p["right"]["b1"][None, :],
        "wr2": p["right"]["w2"], "br2": p["right"]["b2"][None, :],
        "we1": p["edge"]["w1"], "be1": p["edge"]["b1"][None, :],
        "we2": p["edge"]["w2"], "be2": p["edge"]["b2"][None, :],
        "w1f": p["final"]["w1"], "b1f": p["final"]["b1"][None, :],
        "w2f": p["final"]["w2"], "b2f": p["final"]["b2"][None, :],
        "w1ot": w1o[:EMB], "w1ob": w1o[EMB:], "b1o": p["out"]["b1"][None, :],
        "w2o": p["out"]["w2"], "b2o": p["out"]["b2"][None, :],
    }


def kernel(x_constraints, x_variables, edge_index, edge_attr,
           x_constraints_batch, x_variables_batch, params):
    N = x_constraints.shape[0]
    E = edge_attr.shape[0]
    nch = E // NW // EK
    cons_idx = jnp.reshape(edge_index[0], (NW, nch, EK))
    var_idx = jnp.reshape(edge_index[1], (NW, nch, EK))

    convs = [_unpack_conv(params["v_to_c"][0]), _unpack_conv(params["c_to_v"][0]),
             _unpack_conv(params["v_to_c"][1]), _unpack_conv(params["c_to_v"][1])]
    gns = [params["gn_v_to_c"][0], params["gn_c_to_v"][0],
           params["gn_v_to_c"][1], params["gn_c_to_v"][1]]

    zeros_ne = jnp.zeros((N, EMB), _f32)
    cb_f = x_constraints_batch.astype(_f32)[:, None]
    vb_f = x_variables_batch.astype(_f32)[:, None]
    eattr_bf = edge_attr.astype(jnp.bfloat16)

    def conv(f, right, left, dst, src):
        hi, hj = _node_hij(right, left, f["wl1"], f["bl1"], f["wl2"], f["bl2"],
                           f["wr1"], f["br1"], f["wr2"], f["br2"])
        u = _sc_gather(hi, hj, dst, src)
        msg = _msg(u, eattr_bf, f["we1"], f["be1"], f["we2"], f["be2"],
                   f["w1f"], f["b1f"], f["w2f"], f["b2f"])
        s2 = _sc_scatter(msg, dst, zeros_ne)
        return _post(s2, right, f["w1ot"], f["w1ob"], f["b1o"],
                     f["w2o"], f["b2o"])

    xc, xv = x_constraints, x_variables
    for i in range(DEPTH):
        xc = conv(convs[2 * i], xc, xv, cons_idx, var_idx)
        xv = conv(convs[2 * i + 1], xv, xc, var_idx, cons_idx)
        g = gns[2 * i]
        xc = _gn(xc, cb_f, g["weight"][None, :], g["bias"][None, :],
                 g["mean_scale"][None, :])
        g = gns[2 * i + 1]
        xv = _gn(xv, vb_f, g["weight"][None, :], g["bias"][None, :],
                 g["mean_scale"][None, :])
    return (xc, xv)


# speedup vs baseline: 1.4522x; 1.4358x over previous
import jax
import jax.numpy as jnp
from jax import lax
from jax.experimental import pallas as pl
from jax.experimental.pallas import tpu as pltpu
from jax.experimental.pallas import tpu_sc as plsc

EMB = 128
NG = 32
DEPTH = 2
NSC = 2
NSUB = 16
NW = NSC * NSUB
EK = 80
LL = 16
BE = 1000
BN = 2000

_f32 = jnp.float32


def _bdot(x, w):
    return jnp.dot(x.astype(jnp.bfloat16), w.astype(jnp.bfloat16),
                   preferred_element_type=_f32)


def _hdot(x, w):
    return jnp.dot(x, w, preferred_element_type=_f32,
                   precision=lax.Precision.HIGHEST)


def _edge_he_body(e_ref, w1_ref, b1_ref, w2_ref, b2_ref, o_ref):
    x = e_ref[...]
    for i in range(w1_ref.shape[0]):
        h = jnp.maximum(_bdot(x, w1_ref[i]) + b1_ref[i], 0.0)
        o_ref[i] = _bdot(h, w2_ref[i]) + b2_ref[i]


def _edge_he(eattr, w1s, b1s, w2s, b2s):
    E = eattr.shape[0]
    F = w1s.shape[0]
    mats = pl.BlockSpec((F, EMB, EMB), lambda i: (0, 0, 0))
    vecs = pl.BlockSpec((F, 1, EMB), lambda i: (0, 0, 0))
    return pl.pallas_call(
        _edge_he_body,
        grid=(E // BE,),
        in_specs=[pl.BlockSpec((BE, EMB), lambda i: (i, 0)),
                  mats, vecs, mats, vecs],
        out_specs=pl.BlockSpec((F, BE, EMB), lambda i: (0, i, 0)),
        out_shape=jax.ShapeDtypeStruct((F, E, EMB), _f32),
    )(eattr, w1s, b1s, w2s, b2s)


def _node_hij_body(r_ref, l_ref, wl1, bl1, wl2, bl2, wr1, br1, wr2, br2,
                   hi_ref, hj_ref):
    h = jnp.maximum(_bdot(r_ref[...], wl1[...]) + bl1[...], 0.0)
    hi_ref[...] = _bdot(h, wl2[...]) + bl2[...]
    h2 = jnp.maximum(_bdot(l_ref[...], wr1[...]) + br1[...], 0.0)
    hj_ref[...] = _bdot(h2, wr2[...]) + br2[...]


def _node_hij(right, left, wl1, bl1, wl2, bl2, wr1, br1, wr2, br2):
    N = right.shape[0]
    mat = pl.BlockSpec((EMB, EMB), lambda i: (0, 0))
    vec = pl.BlockSpec((1, EMB), lambda i: (0, 0))
    blk = pl.BlockSpec((BN, EMB), lambda i: (i, 0))
    return pl.pallas_call(
        _node_hij_body,
        grid=(N // BN,),
        in_specs=[blk, blk, mat, vec, mat, vec, mat, vec, mat, vec],
        out_specs=(blk, blk),
        out_shape=(jax.ShapeDtypeStruct((N, EMB), _f32),
                   jax.ShapeDtypeStruct((N, EMB), _f32)),
    )(right, left, wl1, bl1, wl2, bl2, wr1, br1, wr2, br2)


def _msg_body(u_ref, e_ref, we1, be1, we2, be2, w1f, b1f, w2f, b2f, o_ref):
    x = e_ref[...]
    h = jnp.maximum(_bdot(x, we1[...]) + be1[...], 0.0)
    he = _bdot(h, we2[...]) + be2[...]
    t = u_ref[...] + he
    h2 = jnp.maximum(_bdot(t, w1f[...]) + b1f[...], 0.0)
    o_ref[...] = _bdot(h2, w2f[...]) + b2f[...]


def _msg(u, eattr_bf, we1, be1, we2, be2, w1f, b1f, w2f, b2f):
    E = u.shape[0]
    bem = 1600
    mat = pl.BlockSpec((EMB, EMB), lambda i: (0, 0))
    vec = pl.BlockSpec((1, EMB), lambda i: (0, 0))
    blk = pl.BlockSpec((bem, EMB), lambda i: (i, 0))
    return pl.pallas_call(
        _msg_body,
        grid=(E // bem,),
        in_specs=[blk, blk, mat, vec, mat, vec, mat, vec, mat, vec],
        out_specs=blk,
        out_shape=jax.ShapeDtypeStruct((E, EMB), _f32),
    )(u, eattr_bf, we1, be1, we2, be2, w1f, b1f, w2f, b2f)


def _post_body(s_ref, r_ref, w1ot, w1ob, b1o, w2o, b2o, o_ref):
    agg = s_ref[0] + s_ref[1]
    pre = _bdot(agg, w1ot[...]) + _bdot(r_ref[...], w1ob[...]) + b1o[...]
    h = jnp.maximum(pre, 0.0)
    o_ref[...] = _bdot(h, w2o[...]) + b2o[...]


def _post(s2, right, w1ot, w1ob, b1o, w2o, b2o):
    N = right.shape[0]
    mat = pl.BlockSpec((EMB, EMB), lambda i: (0, 0))
    vec = pl.BlockSpec((1, EMB), lambda i: (0, 0))
    blk = pl.BlockSpec((BN, EMB), lambda i: (i, 0))
    sblk = pl.BlockSpec((NSC, BN, EMB), lambda i: (0, i, 0))
    return pl.pallas_call(
        _post_body,
        grid=(N // BN,),
        in_specs=[sblk, blk, mat, mat, vec, mat, vec],
        out_specs=blk,
        out_shape=jax.ShapeDtypeStruct((N, EMB), _f32),
    )(s2, right, w1ot, w1ob, b1o, w2o, b2o)


def _gn_stats_body(x_ref, b_ref, s1_ref, s2_ref, c_ref):
    @pl.when(pl.program_id(0) == 0)
    def _():
        s1_ref[...] = jnp.zeros_like(s1_ref)
        s2_ref[...] = jnp.zeros_like(s2_ref)
        c_ref[...] = jnp.zeros_like(c_ref)

    x = x_ref[...]
    g = lax.broadcasted_iota(jnp.int32, (BN, NG), 1).astype(_f32)
    m = (b_ref[...] == g).astype(_f32)
    dn = (((0,), (0,)), ((), ()))
    s1_ref[...] += lax.dot_general(m, x, dn, preferred_element_type=_f32,
                                   precision=lax.Precision.HIGHEST)
    s2_ref[...] += lax.dot_general(m, x * x, dn, preferred_element_type=_f32,
                                   precision=lax.Precision.HIGHEST)
    c_ref[...] += jnp.sum(m, axis=0)[:, None] * jnp.ones((1, EMB), _f32)


def _gn_norm_body(x_ref, b_ref, s1_ref, s2_ref, c_ref, w, bias, msc, o_ref):
    cnt = jnp.maximum(c_ref[...], 1.0)
    mean = s1_ref[...] / cnt
    ms = mean * msc[...]
    var = s2_ref[...] / cnt - 2.0 * ms * mean + ms * ms
    rstd = lax.rsqrt(var + 1e-5)
    g = lax.broadcasted_iota(jnp.int32, (BN, NG), 1).astype(_f32)
    m = (b_ref[...] == g).astype(_f32)
    mloc = _hdot(m, ms)
    rloc = _hdot(m, rstd)
    o_ref[...] = w[...] * (x_ref[...] - mloc) * rloc + bias[...]


def _gn(x, batch_f, w, bias, msc):
    N = x.shape[0]
    blk = pl.BlockSpec((BN, EMB), lambda i: (i, 0))
    bblk = pl.BlockSpec((BN, 1), lambda i: (i, 0))
    stat = pl.BlockSpec((NG, EMB), lambda i: (0, 0))
    vec = pl.BlockSpec((1, EMB), lambda i: (0, 0))
    s1, s2, cnt = pl.pallas_call(
        _gn_stats_body,
        grid=(N // BN,),
        in_specs=[blk, bblk],
        out_specs=(stat, stat, stat),
        out_shape=(jax.ShapeDtypeStruct((NG, EMB), _f32),) * 3,
    )(x, batch_f)
    return pl.pallas_call(
        _gn_norm_body,
        grid=(N // BN,),
        in_specs=[blk, bblk, stat, stat, stat, vec, vec, vec],
        out_specs=blk,
        out_shape=jax.ShapeDtypeStruct((N, EMB), _f32),
    )(x, batch_f, s1, s2, cnt, w, bias, msc)


def _sc_gather(hi, hj, dst3, src3):
    N = hi.shape[0]
    nch = dst3.shape[1]
    ept = nch * EK
    E = NW * ept
    mesh = plsc.VectorSubcoreMesh(core_axis_name="c", subcore_axis_name="s")

    def body(hi_hbm, hj_hbm, dst_hbm, src_hbm, t_hbm,
             dst_v, src_v, bufa, bufb, sga, sgb, st):
        ci = lax.axis_index("c")
        si = lax.axis_index("s")
        wid = si * NSC + ci
        pltpu.sync_copy(dst_hbm.at[wid], dst_v)
        pltpu.sync_copy(src_hbm.at[wid], src_v)

        def fire(j, b):
            return (pltpu.async_copy(hi_hbm.at[dst_v.at[j]], bufa[b], sga[b]),
                    pltpu.async_copy(hj_hbm.at[src_v.at[j]], bufb[b], sgb[b]))

        def compute(b):
            def row(r, rc):
                for l in range(EMB // LL):
                    sl = pl.ds(l * LL, LL)
                    bufa[b][r, sl] = bufa[b][r, sl] + bufb[b][r, sl]
                return rc
            lax.fori_loop(0, EK, row, 0)

        def write_t(j, b):
            base = wid * ept + j * EK
            return pltpu.async_copy(bufa[b], t_hbm.at[pl.ds(base, EK)], st[b])

        def pair(jj, carry):
            j0 = 2 * jj
            j1 = j0 + 1
            g0 = fire(j0, 0)
            g1 = fire(j1, 1)
            for cp in g0:
                cp.wait()
            compute(0)
            t0 = write_t(j0, 0)
            for cp in g1:
                cp.wait()
            compute(1)
            t1 = write_t(j1, 1)
            t0.wait()
            t1.wait()
            return carry
        lax.fori_loop(0, nch // 2, pair, 0)
        if nch % 2:
            g = fire(nch - 1, 0)
            for cp in g:
                cp.wait()
            compute(0)
            write_t(nch - 1, 0).wait()

    kfn = pl.kernel(
        body,
        out_type=jax.ShapeDtypeStruct((E, EMB), _f32),
        mesh=mesh,
        scratch_types=[
            pltpu.VMEM((nch, EK), jnp.int32),
            pltpu.VMEM((nch, EK), jnp.int32),
            [pltpu.VMEM((EK, EMB), _f32)] * 2,
            [pltpu.VMEM((EK, EMB), _f32)] * 2,
            [pltpu.SemaphoreType.DMA] * 2,
            [pltpu.SemaphoreType.DMA] * 2,
            [pltpu.SemaphoreType.DMA] * 2,
        ],
    )
    return kfn(hi, hj, dst3, src3)


def _sc_scatter(msg, dst3, zeros_ne):
    N = zeros_ne.shape[0]
    nch = dst3.shape[1]
    ept = nch * EK
    rps = (N // NSUB) // 8 * 8
    rem = N - NSUB * rps
    rbase = NSUB * rps
    mesh = plsc.VectorSubcoreMesh(core_axis_name="c", subcore_axis_name="s")

    def body(msg_hbm, dst_hbm, z_hbm, out_hbm, s_s, dst_v, bufm, sgm, ssc):
        ci = lax.axis_index("c")
        si = lax.axis_index("s")
        wid = si * NSC + ci
        pltpu.sync_copy(dst_hbm.at[wid], dst_v)
        pltpu.sync_copy(z_hbm.at[pl.ds(si * rps, rps)],
                        s_s.at[pl.ds(si * rps, rps)])
        if rem:
            @pl.when(si == 0)
            def _():
                pltpu.sync_copy(z_hbm.at[pl.ds(rbase, rem)],
                                s_s.at[pl.ds(rbase, rem)])
        plsc.subcore_barrier()

        def read_msg(j, b):
            base = wid * ept + j * EK
            return pltpu.async_copy(msg_hbm.at[pl.ds(base, EK)], bufm[b],
                                    sgm[b])

        nbuf = 3

        def quad(jj, carry):
            j0 = nbuf * jj
            reads = [read_msg(j0 + b, b) for b in range(nbuf)]
            scats = []
            for b in range(nbuf):
                reads[b].wait()
                scats.append(pltpu.async_copy(bufm[b],
                                              s_s.at[dst_v.at[j0 + b]],
                                              ssc[b], add=True))
            for cp in scats:
                cp.wait()
            return carry
        lax.fori_loop(0, nch // nbuf, quad, 0)
        for j in range(nch - nch % nbuf, nch):
            read_msg(j, 0).wait()
            pltpu.sync_copy(bufm[0], s_s.at[dst_v.at[j]], add=True)
        plsc.subcore_barrier()
        pltpu.sync_copy(s_s.at[pl.ds(si * rps, rps)],
                        out_hbm.at[ci, pl.ds(si * rps, rps)])
        if rem:
            @pl.when(si == 0)
            def _():
                pltpu.sync_copy(s_s.at[pl.ds(rbase, rem)],
                                out_hbm.at[ci, pl.ds(rbase, rem)])

    kfn = pl.kernel(
        body,
        out_type=jax.ShapeDtypeStruct((NSC, N, EMB), _f32),
        mesh=mesh,
        scratch_types=[
            pltpu.VMEM_SHARED((N, EMB), _f32),
            pltpu.VMEM((nch, EK), jnp.int32),
            [pltpu.VMEM((EK, EMB), _f32)] * 3,
            [pltpu.SemaphoreType.DMA] * 3,
            [pltpu.SemaphoreType.DMA] * 3,
        ],
    )
    return kfn(msg, dst3, zeros_ne)


def _unpack_conv(p):
    w1o = p["out"]["w1"]
    return {
        "wl1": p["left"]["w1"], "bl1": p["left"]["b1"][None, :],
        "wl2": p["left"]["w2"], "bl2": p["left"]["b2"][None, :],
        "wr1": p["right"]["w1"], "br1": p["right"]["b1"][None, :],
        "wr2": p["right"]["w2"], "br2": p["right"]["b2"][None, :],
        "we1": p["edge"]["w1"], "be1": p["edge"]["b1"][None, :],
        "we2": p["edge"]["w2"], "be2": p["edge"]["b2"][None, :],
        "w1f": p["final"]["w1"], "b1f": p["final"]["b1"][None, :],
        "w2f": p["final"]["w2"], "b2f": p["final"]["b2"][None, :],
        "w1ot": w1o[:EMB], "w1ob": w1o[EMB:], "b1o": p["out"]["b1"][None, :],
        "w2o": p["out"]["w2"], "b2o": p["out"]["b2"][None, :],
    }


def kernel(x_constraints, x_variables, edge_index, edge_attr,
           x_constraints_batch, x_variables_batch, params):
    N = x_constraints.shape[0]
    E = edge_attr.shape[0]
    nch = E // NW // EK
    cons_idx = jnp.reshape(edge_index[0], (NW, nch, EK))
    var_idx = jnp.reshape(edge_index[1], (NW, nch, EK))

    convs = [_unpack_conv(params["v_to_c"][0]), _unpack_conv(params["c_to_v"][0]),
             _unpack_conv(params["v_to_c"][1]), _unpack_conv(params["c_to_v"][1])]
    gns = [params["gn_v_to_c"][0], params["gn_c_to_v"][0],
           params["gn_v_to_c"][1], params["gn_c_to_v"][1]]

    zeros_ne = jnp.zeros((N, EMB), _f32)
    cb_f = x_constraints_batch.astype(_f32)[:, None]
    vb_f = x_variables_batch.astype(_f32)[:, None]
    eattr_bf = edge_attr.astype(jnp.bfloat16)

    def conv(f, right, left, dst, src):
        hi, hj = _node_hij(right, left, f["wl1"], f["bl1"], f["wl2"], f["bl2"],
                           f["wr1"], f["br1"], f["wr2"], f["br2"])
        u = _sc_gather(hi, hj, dst, src)
        msg = _msg(u, eattr_bf, f["we1"], f["be1"], f["we2"], f["be2"],
                   f["w1f"], f["b1f"], f["w2f"], f["b2f"])
        s2 = _sc_scatter(msg, dst, zeros_ne)
        return _post(s2, right, f["w1ot"], f["w1ob"], f["b1o"],
                     f["w2o"], f["b2o"])

    xc, xv = x_constraints, x_variables
    for i in range(DEPTH):
        xc = conv(convs[2 * i], xc, xv, cons_idx, var_idx)
        xv = conv(convs[2 * i + 1], xv, xc, var_idx, cons_idx)
        g = gns[2 * i]
        xc = _gn(xc, cb_f, g["weight"][None, :], g["bias"][None, :],
                 g["mean_scale"][None, :])
        g = gns[2 * i + 1]
        xv = _gn(xv, vb_f, g["weight"][None, :], g["bias"][None, :],
                 g["mean_scale"][None, :])
    return (xc, xv)


# speedup vs baseline: 1.6015x; 1.1028x over previous
import jax
import jax.numpy as jnp
from jax import lax
from jax.experimental import pallas as pl
from jax.experimental.pallas import tpu as pltpu
from jax.experimental.pallas import tpu_sc as plsc

EMB = 128
NG = 32
DEPTH = 2
NSC = 2
NSUB = 16
NW = NSC * NSUB
EK = 80
LL = 16
BE = 1000
BN = 5000

_f32 = jnp.float32


def _bdot(x, w):
    return jnp.dot(x.astype(jnp.bfloat16), w.astype(jnp.bfloat16),
                   preferred_element_type=_f32)


def _hdot(x, w):
    return jnp.dot(x, w, preferred_element_type=_f32,
                   precision=lax.Precision.HIGHEST)


def _edge_he_body(e_ref, w1_ref, b1_ref, w2_ref, b2_ref, o_ref):
    x = e_ref[...]
    for i in range(w1_ref.shape[0]):
        h = jnp.maximum(_bdot(x, w1_ref[i]) + b1_ref[i], 0.0)
        o_ref[i] = _bdot(h, w2_ref[i]) + b2_ref[i]


def _edge_he(eattr, w1s, b1s, w2s, b2s):
    E = eattr.shape[0]
    F = w1s.shape[0]
    mats = pl.BlockSpec((F, EMB, EMB), lambda i: (0, 0, 0))
    vecs = pl.BlockSpec((F, 1, EMB), lambda i: (0, 0, 0))
    return pl.pallas_call(
        _edge_he_body,
        grid=(E // BE,),
        in_specs=[pl.BlockSpec((BE, EMB), lambda i: (i, 0)),
                  mats, vecs, mats, vecs],
        out_specs=pl.BlockSpec((F, BE, EMB), lambda i: (0, i, 0)),
        out_shape=jax.ShapeDtypeStruct((F, E, EMB), _f32),
    )(eattr, w1s, b1s, w2s, b2s)


def _node_hij_body(r_ref, l_ref, wl1, bl1, wl2, bl2, wr1, br1, wr2, br2,
                   hi_ref, hj_ref):
    h = jnp.maximum(_bdot(r_ref[...], wl1[...]) + bl1[...], 0.0)
    hi_ref[...] = _bdot(h, wl2[...]) + bl2[...]
    h2 = jnp.maximum(_bdot(l_ref[...], wr1[...]) + br1[...], 0.0)
    hj_ref[...] = _bdot(h2, wr2[...]) + br2[...]


def _node_hij(right, left, wl1, bl1, wl2, bl2, wr1, br1, wr2, br2):
    N = right.shape[0]
    mat = pl.BlockSpec((EMB, EMB), lambda i: (0, 0))
    vec = pl.BlockSpec((1, EMB), lambda i: (0, 0))
    blk = pl.BlockSpec((BN, EMB), lambda i: (i, 0))
    return pl.pallas_call(
        _node_hij_body,
        grid=(N // BN,),
        in_specs=[blk, blk, mat, vec, mat, vec, mat, vec, mat, vec],
        out_specs=(blk, blk),
        out_shape=(jax.ShapeDtypeStruct((N, EMB), _f32),
                   jax.ShapeDtypeStruct((N, EMB), _f32)),
    )(right, left, wl1, bl1, wl2, bl2, wr1, br1, wr2, br2)


def _msg_body(u_ref, e_ref, we1, be1, we2, be2, w1f, b1f, w2f, b2f, o_ref):
    x = e_ref[...]
    h = jnp.maximum(_bdot(x, we1[...]) + be1[...], 0.0)
    he = _bdot(h, we2[...]) + be2[...]
    t = u_ref[...] + he
    h2 = jnp.maximum(_bdot(t, w1f[...]) + b1f[...], 0.0)
    o_ref[...] = _bdot(h2, w2f[...]) + b2f[...]


def _msg(u, eattr_bf, we1, be1, we2, be2, w1f, b1f, w2f, b2f):
    E = u.shape[0]
    bem = 3200
    mat = pl.BlockSpec((EMB, EMB), lambda i: (0, 0))
    vec = pl.BlockSpec((1, EMB), lambda i: (0, 0))
    blk = pl.BlockSpec((bem, EMB), lambda i: (i, 0))
    return pl.pallas_call(
        _msg_body,
        grid=(E // bem,),
        in_specs=[blk, blk, mat, vec, mat, vec, mat, vec, mat, vec],
        out_specs=blk,
        out_shape=jax.ShapeDtypeStruct((E, EMB), _f32),
    )(u, eattr_bf, we1, be1, we2, be2, w1f, b1f, w2f, b2f)


def _post_body(s_ref, r_ref, w1ot, w1ob, b1o, w2o, b2o, o_ref):
    agg = s_ref[0] + s_ref[1]
    pre = _bdot(agg, w1ot[...]) + _bdot(r_ref[...], w1ob[...]) + b1o[...]
    h = jnp.maximum(pre, 0.0)
    o_ref[...] = _bdot(h, w2o[...]) + b2o[...]


def _post(s2, right, w1ot, w1ob, b1o, w2o, b2o):
    N = right.shape[0]
    mat = pl.BlockSpec((EMB, EMB), lambda i: (0, 0))
    vec = pl.BlockSpec((1, EMB), lambda i: (0, 0))
    blk = pl.BlockSpec((BN, EMB), lambda i: (i, 0))
    sblk = pl.BlockSpec((NSC, BN, EMB), lambda i: (0, i, 0))
    return pl.pallas_call(
        _post_body,
        grid=(N // BN,),
        in_specs=[sblk, blk, mat, mat, vec, mat, vec],
        out_specs=blk,
        out_shape=jax.ShapeDtypeStruct((N, EMB), _f32),
    )(s2, right, w1ot, w1ob, b1o, w2o, b2o)


def _gn_stats_body(x_ref, b_ref, s1_ref, s2_ref, c_ref):
    @pl.when(pl.program_id(0) == 0)
    def _():
        s1_ref[...] = jnp.zeros_like(s1_ref)
        s2_ref[...] = jnp.zeros_like(s2_ref)
        c_ref[...] = jnp.zeros_like(c_ref)

    x = x_ref[...]
    g = lax.broadcasted_iota(jnp.int32, (BN, NG), 1).astype(_f32)
    m = (b_ref[...] == g).astype(_f32)
    dn = (((0,), (0,)), ((), ()))
    s1_ref[...] += lax.dot_general(m, x, dn, preferred_element_type=_f32,
                                   precision=lax.Precision.HIGHEST)
    s2_ref[...] += lax.dot_general(m, x * x, dn, preferred_element_type=_f32,
                                   precision=lax.Precision.HIGHEST)
    c_ref[...] += jnp.sum(m, axis=0)[:, None] * jnp.ones((1, EMB), _f32)


def _gn_norm_body(x_ref, b_ref, s1_ref, s2_ref, c_ref, w, bias, msc, o_ref):
    cnt = jnp.maximum(c_ref[...], 1.0)
    mean = s1_ref[...] / cnt
    ms = mean * msc[...]
    var = s2_ref[...] / cnt - 2.0 * ms * mean + ms * ms
    rstd = lax.rsqrt(var + 1e-5)
    g = lax.broadcasted_iota(jnp.int32, (BN, NG), 1).astype(_f32)
    m = (b_ref[...] == g).astype(_f32)
    mloc = _hdot(m, ms)
    rloc = _hdot(m, rstd)
    o_ref[...] = w[...] * (x_ref[...] - mloc) * rloc + bias[...]


def _gn(x, batch_f, w, bias, msc):
    N = x.shape[0]
    blk = pl.BlockSpec((BN, EMB), lambda i: (i, 0))
    bblk = pl.BlockSpec((BN, 1), lambda i: (i, 0))
    stat = pl.BlockSpec((NG, EMB), lambda i: (0, 0))
    vec = pl.BlockSpec((1, EMB), lambda i: (0, 0))
    s1, s2, cnt = pl.pallas_call(
        _gn_stats_body,
        grid=(N // BN,),
        in_specs=[blk, bblk],
        out_specs=(stat, stat, stat),
        out_shape=(jax.ShapeDtypeStruct((NG, EMB), _f32),) * 3,
    )(x, batch_f)
    return pl.pallas_call(
        _gn_norm_body,
        grid=(N // BN,),
        in_specs=[blk, bblk, stat, stat, stat, vec, vec, vec],
        out_specs=blk,
        out_shape=jax.ShapeDtypeStruct((N, EMB), _f32),
    )(x, batch_f, s1, s2, cnt, w, bias, msc)


def _sc_gather(hi, hj, dst3, src3):
    N = hi.shape[0]
    nch = dst3.shape[1]
    ept = nch * EK
    E = NW * ept
    mesh = plsc.VectorSubcoreMesh(core_axis_name="c", subcore_axis_name="s")

    def body(hi_hbm, hj_hbm, dst_hbm, src_hbm, t_hbm,
             dst_v, src_v, bufa, bufb, sga, sgb, st):
        ci = lax.axis_index("c")
        si = lax.axis_index("s")
        wid = si * NSC + ci
        pltpu.sync_copy(dst_hbm.at[wid], dst_v)
        pltpu.sync_copy(src_hbm.at[wid], src_v)

        def fire(j, b):
            return (pltpu.async_copy(hi_hbm.at[dst_v.at[j]], bufa[b], sga[b]),
                    pltpu.async_copy(hj_hbm.at[src_v.at[j]], bufb[b], sgb[b]))

        def compute(b):
            def row(r, rc):
                for l in range(EMB // LL):
                    sl = pl.ds(l * LL, LL)
                    bufa[b][r, sl] = bufa[b][r, sl] + bufb[b][r, sl]
                return rc
            lax.fori_loop(0, EK, row, 0)

        def write_t(j, b):
            base = wid * ept + j * EK
            return pltpu.async_copy(bufa[b], t_hbm.at[pl.ds(base, EK)], st[b])

        def pair(jj, carry):
            j0 = 2 * jj
            j1 = j0 + 1
            g0 = fire(j0, 0)
            g1 = fire(j1, 1)
            for cp in g0:
                cp.wait()
            compute(0)
            t0 = write_t(j0, 0)
            for cp in g1:
                cp.wait()
            compute(1)
            t1 = write_t(j1, 1)
            t0.wait()
            t1.wait()
            return carry
        lax.fori_loop(0, nch // 2, pair, 0)
        if nch % 2:
            g = fire(nch - 1, 0)
            for cp in g:
                cp.wait()
            compute(0)
            write_t(nch - 1, 0).wait()

    kfn = pl.kernel(
        body,
        out_type=jax.ShapeDtypeStruct((E, EMB), _f32),
        mesh=mesh,
        scratch_types=[
            pltpu.VMEM((nch, EK), jnp.int32),
            pltpu.VMEM((nch, EK), jnp.int32),
            [pltpu.VMEM((EK, EMB), _f32)] * 2,
            [pltpu.VMEM((EK, EMB), _f32)] * 2,
            [pltpu.SemaphoreType.DMA] * 2,
            [pltpu.SemaphoreType.DMA] * 2,
            [pltpu.SemaphoreType.DMA] * 2,
        ],
    )
    return kfn(hi, hj, dst3, src3)


def _sc_scatter(msg, dst3, zeros_ne):
    N = zeros_ne.shape[0]
    nch = dst3.shape[1]
    ept = nch * EK
    rps = (N // NSUB) // 8 * 8
    rem = N - NSUB * rps
    rbase = NSUB * rps
    mesh = plsc.VectorSubcoreMesh(core_axis_name="c", subcore_axis_name="s")

    def body(msg_hbm, dst_hbm, z_hbm, out_hbm, s_s, dst_v, bufm, sgm, ssc):
        ci = lax.axis_index("c")
        si = lax.axis_index("s")
        wid = si * NSC + ci
        pltpu.sync_copy(dst_hbm.at[wid], dst_v)
        pltpu.sync_copy(z_hbm.at[pl.ds(si * rps, rps)],
                        s_s.at[pl.ds(si * rps, rps)])
        if rem:
            @pl.when(si == 0)
            def _():
                pltpu.sync_copy(z_hbm.at[pl.ds(rbase, rem)],
                                s_s.at[pl.ds(rbase, rem)])
        plsc.subcore_barrier()

        def read_msg(j, b):
            base = wid * ept + j * EK
            return pltpu.async_copy(msg_hbm.at[pl.ds(base, EK)], bufm[b],
                                    sgm[b])

        nbuf = 3

        def quad(jj, carry):
            j0 = nbuf * jj
            reads = [read_msg(j0 + b, b) for b in range(nbuf)]
            scats = []
            for b in range(nbuf):
                reads[b].wait()
                scats.append(pltpu.async_copy(bufm[b],
                                              s_s.at[dst_v.at[j0 + b]],
                                              ssc[b], add=True))
            for cp in scats:
                cp.wait()
            return carry
        lax.fori_loop(0, nch // nbuf, quad, 0)
        for j in range(nch - nch % nbuf, nch):
            read_msg(j, 0).wait()
            pltpu.sync_copy(bufm[0], s_s.at[dst_v.at[j]], add=True)
        plsc.subcore_barrier()
        pltpu.sync_copy(s_s.at[pl.ds(si * rps, rps)],
                        out_hbm.at[ci, pl.ds(si * rps, rps)])
        if rem:
            @pl.when(si == 0)
            def _():
                pltpu.sync_copy(s_s.at[pl.ds(rbase, rem)],
                                out_hbm.at[ci, pl.ds(rbase, rem)])

    kfn = pl.kernel(
        body,
        out_type=jax.ShapeDtypeStruct((NSC, N, EMB), _f32),
        mesh=mesh,
        scratch_types=[
            pltpu.VMEM_SHARED((N, EMB), _f32),
            pltpu.VMEM((nch, EK), jnp.int32),
            [pltpu.VMEM((EK, EMB), _f32)] * 3,
            [pltpu.SemaphoreType.DMA] * 3,
            [pltpu.SemaphoreType.DMA] * 3,
        ],
    )
    return kfn(msg, dst3, zeros_ne)


def _unpack_conv(p):
    w1o = p["out"]["w1"]
    return {
        "wl1": p["left"]["w1"], "bl1": p["left"]["b1"][None, :],
        "wl2": p["left"]["w2"], "bl2": p["left"]["b2"][None, :],
        "wr1": p["right"]["w1"], "br1": p["right"]["b1"][None, :],
        "wr2": p["right"]["w2"], "br2": p["right"]["b2"][None, :],
        "we1": p["edge"]["w1"], "be1": p["edge"]["b1"][None, :],
        "we2": p["edge"]["w2"], "be2": p["edge"]["b2"][None, :],
        "w1f": p["final"]["w1"], "b1f": p["final"]["b1"][None, :],
        "w2f": p["final"]["w2"], "b2f": p["final"]["b2"][None, :],
        "w1ot": w1o[:EMB], "w1ob": w1o[EMB:], "b1o": p["out"]["b1"][None, :],
        "w2o": p["out"]["w2"], "b2o": p["out"]["b2"][None, :],
    }


def kernel(x_constraints, x_variables, edge_index, edge_attr,
           x_constraints_batch, x_variables_batch, params):
    N = x_constraints.shape[0]
    E = edge_attr.shape[0]
    nch = E // NW // EK
    cons_idx = jnp.reshape(edge_index[0], (NW, nch, EK))
    var_idx = jnp.reshape(edge_index[1], (NW, nch, EK))

    convs = [_unpack_conv(params["v_to_c"][0]), _unpack_conv(params["c_to_v"][0]),
             _unpack_conv(params["v_to_c"][1]), _unpack_conv(params["c_to_v"][1])]
    gns = [params["gn_v_to_c"][0], params["gn_c_to_v"][0],
           params["gn_v_to_c"][1], params["gn_c_to_v"][1]]

    zeros_ne = jnp.zeros((N, EMB), _f32)
    cb_f = x_constraints_batch.astype(_f32)[:, None]
    vb_f = x_variables_batch.astype(_f32)[:, None]
    eattr_bf = edge_attr.astype(jnp.bfloat16)

    def conv(f, right, left, dst, src):
        hi, hj = _node_hij(right, left, f["wl1"], f["bl1"], f["wl2"], f["bl2"],
                           f["wr1"], f["br1"], f["wr2"], f["br2"])
        u = _sc_gather(hi, hj, dst, src)
        msg = _msg(u, eattr_bf, f["we1"], f["be1"], f["we2"], f["be2"],
                   f["w1f"], f["b1f"], f["w2f"], f["b2f"])
        s2 = _sc_scatter(msg, dst, zeros_ne)
        return _post(s2, right, f["w1ot"], f["w1ob"], f["b1o"],
                     f["w2o"], f["b2o"])

    xc, xv = x_constraints, x_variables
    for i in range(DEPTH):
        xc = conv(convs[2 * i], xc, xv, cons_idx, var_idx)
        xv = conv(convs[2 * i + 1], xv, xc, var_idx, cons_idx)
        g = gns[2 * i]
        xc = _gn(xc, cb_f, g["weight"][None, :], g["bias"][None, :],
                 g["mean_scale"][None, :])
        g = gns[2 * i + 1]
        xv = _gn(xv, vb_f, g["weight"][None, :], g["bias"][None, :],
                 g["mean_scale"][None, :])
    return (xc, xv)
